# baseline (speedup 1.0000x reference)
import functools

import jax
import jax.numpy as jnp
from jax import lax
from jax.experimental import pallas as pl
from jax.experimental.pallas import tpu as pltpu
from jax.experimental.pallas import tpu_sc as plsc

B = 16384
F = 26
V = 1000000

NC = 2
NS = 16
NW = NC * NS
BPW = B // NW
LANES = 16
ROWLEN = 128
GF = 8
VPAD = 1000064
CTILES = VPAD // ROWLEN
GROWS64 = GF * CTILES * 8
CHUNK = 128
NCHUNK = BPW // CHUNK
VPC = CHUNK // LANES
DEPTH = 4

GROUPS = (8, 8, 8, 2)


def _make_body(nf, has_prev):
    ngather = nf * NCHUNK

    def body(idx_hbm, tab_hbm, aux_hbm, out_hbm, idx_v, rowid_v, rows_v,
             out_v, bias_v, sem):
        wid = lax.axis_index("s") * NC + lax.axis_index("c")
        base = wid * BPW

        pltpu.sync_copy(idx_hbm.at[:, pl.ds(base, BPW)], idx_v)

        if has_prev:
            pltpu.sync_copy(aux_hbm.at[pl.ds(base, BPW)], out_v)
        else:
            pltpu.sync_copy(aux_hbm, bias_v)
            b_vec = bias_v[...]

            def init(c, carry):
                out_v[pl.ds(c * LANES, LANES)] = b_vec
                return carry

            lax.fori_loop(0, BPW // LANES, init, 0)

        lane_iota = lax.iota(jnp.int32, LANES)

        def fire(j):
            f = j // NCHUNK
            q = j % NCHUNK
            slot = j % DEPTH
            foff = f * 8

            def rid(k, carry2):
                iv = idx_v[f, pl.ds(q * CHUNK + k * LANES, LANES)]
                rowid_v[slot, pl.ds(k * LANES, LANES)] = (
                    ((iv >> 7) << 6) + ((iv >> 4) & 7) + foff)
                return carry2

            lax.fori_loop(0, VPC, rid, 0)
            pltpu.make_async_copy(
                tab_hbm.at[rowid_v.at[slot]],
                rows_v.at[slot],
                sem.at[slot],
            ).start()

        for d in range(DEPTH):
            fire(d)

        def chunk_body(j, carry):
            slot = j % DEPTH
            f = j // NCHUNK
            q = j % NCHUNK
            pltpu.make_async_copy(
                tab_hbm.at[pl.ds(0, CHUNK), :], rows_v.at[slot], sem.at[slot]
            ).wait()

            def sel(k, carry2):
                col = q * CHUNK + k * LANES
                iv = idx_v[f, pl.ds(col, LANES)]
                pos = lane_iota + k * LANES
                v = plsc.load_gather(rows_v.at[slot], [pos, iv & 15])
                acc = out_v[pl.ds(col, LANES)]
                out_v[pl.ds(col, LANES)] = acc + v
                return carry2

            lax.fori_loop(0, VPC, sel, 0)

            @pl.when(j + DEPTH < ngather)
            def _():
                fire(j + DEPTH)

            return carry

        lax.fori_loop(0, ngather, chunk_body, 0)

        pltpu.sync_copy(out_v, out_hbm.at[pl.ds(base, BPW)])

    return body


def _group_call(nf, has_prev, idx_g, tab_g, aux):
    mesh = plsc.VectorSubcoreMesh(core_axis_name="c", subcore_axis_name="s")
    return pl.kernel(
        _make_body(nf, has_prev),
        out_type=jax.ShapeDtypeStruct((B,), jnp.float32),
        mesh=mesh,
        scratch_types=[
            pltpu.VMEM((nf, BPW), jnp.int32),
            pltpu.VMEM((DEPTH, CHUNK), jnp.int32),
            pltpu.VMEM((DEPTH, CHUNK, LANES), jnp.float32),
            pltpu.VMEM((BPW,), jnp.float32),
            pltpu.VMEM((LANES,), jnp.float32),
            pltpu.SemaphoreType.DMA((DEPTH,)),
        ],
        compiler_params=pltpu.CompilerParams(
            needs_layout_passes=False, use_tc_tiling_on_sc=False),
        name=f"lookup_sum_g{nf}_{int(has_prev)}",
    )(idx_g, tab_g, aux)


@jax.jit
def _lookup_sum(idx_t, tables, bias16):
    part = None
    f0 = 0
    for nf in GROUPS:
        tab_pad = jnp.pad(tables[f0:f0 + nf], ((0, GF - nf), (0, VPAD - V)))
        tab_g = (
            tab_pad.reshape(1, GF, CTILES, ROWLEN)
            .transpose(0, 2, 1, 3)
            .reshape(GROWS64, LANES)
        )
        idx_g = idx_t[f0:f0 + nf]
        aux = bias16 if part is None else part
        part = _group_call(nf, part is not None, idx_g, tab_g, aux)
        f0 += nf
    return part


def kernel(indices, tables, bias):
    idx_t = indices.astype(jnp.int32).T
    bias16 = jnp.broadcast_to(bias.astype(jnp.float32), (LANES,))
    return _lookup_sum(idx_t, tables, bias16)


# speedup vs baseline: 1.0161x; 1.0161x over previous
import functools

import jax
import jax.numpy as jnp
from jax import lax
from jax.experimental import pallas as pl
from jax.experimental.pallas import tpu as pltpu
from jax.experimental.pallas import tpu_sc as plsc

B = 16384
F = 26
V = 1000000

NC = 2
NS = 16
NW = NC * NS
BPW = B // NW
LANES = 16
ROWLEN = 128
GF = 8
VPAD = 1000064
CTILES = VPAD // ROWLEN
GROWS64 = GF * CTILES * 8
CHUNK = 128
NCHUNK = BPW // CHUNK
VPC = CHUNK // LANES
DEPTH = 4

GROUPS = (8, 8, 8, 2)


def _make_body(f0, nf, has_prev):
    ngather = nf * NCHUNK

    def body(idx_hbm, tab_hbm, aux_hbm, out_hbm, idx_v, rowid_v, rows_v,
             out_v, bias_v, sem):
        wid = lax.axis_index("s") * NC + lax.axis_index("c")
        base = wid * BPW

        pltpu.sync_copy(idx_hbm.at[pl.ds(f0, nf), pl.ds(base, BPW)], idx_v)

        if has_prev:
            pltpu.sync_copy(aux_hbm.at[pl.ds(base, BPW)], out_v)
        else:
            pltpu.sync_copy(aux_hbm, bias_v)
            b_vec = bias_v[...]

            def init(c, carry):
                out_v[pl.ds(c * LANES, LANES)] = b_vec
                return carry

            lax.fori_loop(0, BPW // LANES, init, 0)

        lane_iota = lax.iota(jnp.int32, LANES)

        def fire(j):
            f = j // NCHUNK
            q = j % NCHUNK
            slot = j % DEPTH
            foff = f * 8

            def rid(k, carry2):
                iv = idx_v[f, pl.ds(q * CHUNK + k * LANES, LANES)]
                rowid_v[slot, pl.ds(k * LANES, LANES)] = (
                    ((iv >> 7) << 6) + ((iv >> 4) & 7) + foff)
                return carry2

            lax.fori_loop(0, VPC, rid, 0)
            pltpu.make_async_copy(
                tab_hbm.at[rowid_v.at[slot]],
                rows_v.at[slot],
                sem.at[slot],
            ).start()

        for d in range(DEPTH):
            fire(d)

        def chunk_body(j, carry):
            slot = j % DEPTH
            f = j // NCHUNK
            q = j % NCHUNK
            pltpu.make_async_copy(
                tab_hbm.at[pl.ds(0, CHUNK), :], rows_v.at[slot], sem.at[slot]
            ).wait()

            def sel(k, carry2):
                col = q * CHUNK + k * LANES
                iv = idx_v[f, pl.ds(col, LANES)]
                pos = lane_iota + k * LANES
                v = plsc.load_gather(rows_v.at[slot], [pos, iv & 15])
                acc = out_v[pl.ds(col, LANES)]
                out_v[pl.ds(col, LANES)] = acc + v
                return carry2

            lax.fori_loop(0, VPC, sel, 0)

            @pl.when(j + DEPTH < ngather)
            def _():
                fire(j + DEPTH)

            return carry

        lax.fori_loop(0, ngather, chunk_body, 0)

        pltpu.sync_copy(out_v, out_hbm.at[pl.ds(base, BPW)])

    return body


def _group_call(f0, nf, has_prev, idx_t, tab_g, aux):
    mesh = plsc.VectorSubcoreMesh(core_axis_name="c", subcore_axis_name="s")
    return pl.kernel(
        _make_body(f0, nf, has_prev),
        out_type=jax.ShapeDtypeStruct((B,), jnp.float32),
        mesh=mesh,
        scratch_types=[
            pltpu.VMEM((nf, BPW), jnp.int32),
            pltpu.VMEM((DEPTH, CHUNK), jnp.int32),
            pltpu.VMEM((DEPTH, CHUNK, LANES), jnp.float32),
            pltpu.VMEM((BPW,), jnp.float32),
            pltpu.VMEM((LANES,), jnp.float32),
            pltpu.SemaphoreType.DMA((DEPTH,)),
        ],
        compiler_params=pltpu.CompilerParams(
            needs_layout_passes=False, use_tc_tiling_on_sc=False),
        name=f"lookup_sum_f{f0}_{nf}",
    )(idx_t, tab_g, aux)


@jax.jit
def _lookup_sum(idx_t, tables, bias16):
    part = None
    f0 = 0
    for nf in GROUPS:
        tab_pad = jnp.pad(tables[f0:f0 + nf], ((0, GF - nf), (0, VPAD - V)))
        tab_g = (
            tab_pad.reshape(1, GF, CTILES, ROWLEN)
            .transpose(0, 2, 1, 3)
            .reshape(GROWS64, LANES)
        )
        aux = bias16 if part is None else part
        part = _group_call(f0, nf, part is not None, idx_t, tab_g, aux)
        f0 += nf
    return part


def kernel(indices, tables, bias):
    idx_t = indices.astype(jnp.int32).T
    bias16 = jnp.broadcast_to(bias.astype(jnp.float32), (LANES,))
    return _lookup_sum(idx_t, tables, bias16)


# speedup vs baseline: 1.4594x; 1.4363x over previous
import functools

import jax
import jax.numpy as jnp
from jax import lax
from jax.experimental import pallas as pl
from jax.experimental.pallas import tpu as pltpu
from jax.experimental.pallas import tpu_sc as plsc

B = 16384
F = 26
V = 1000000

NC = 2
NS = 16
NW = NC * NS
BPW = B // NW
LANES = 16
ROWLEN = 128
GF = 8
VPAD = 1000064
CTILES = VPAD // ROWLEN
GROWS64 = GF * CTILES * 8
CHUNK = 128
NCHUNK = BPW // CHUNK
VPC = CHUNK // LANES
DEPTH = 4

GROUPS = (8, 8, 8, 2)


def _make_body(f0, nf, has_prev):
    ngather = nf * NCHUNK

    def body(idx_hbm, tab_hbm, aux_hbm, out_hbm, idx_v, rowid_v, rows_v,
             out_v, bias_v, sem):
        wid = lax.axis_index("s") * NC + lax.axis_index("c")
        base = wid * BPW

        pltpu.sync_copy(idx_hbm.at[pl.ds(f0, nf), pl.ds(base, BPW)], idx_v)

        if has_prev:
            pltpu.sync_copy(aux_hbm.at[pl.ds(base, BPW)], out_v)
        else:
            pltpu.sync_copy(aux_hbm, bias_v)
            b_vec = bias_v[...]

            def init(c, carry):
                out_v[pl.ds(c * LANES, LANES)] = b_vec
                return carry

            lax.fori_loop(0, BPW // LANES, init, 0)

        lane_iota = lax.iota(jnp.int32, LANES)

        def fire(j):
            f = j // NCHUNK
            q = j % NCHUNK
            slot = j % DEPTH
            foff = f * 8

            def rid(k, carry2):
                iv = idx_v[f, pl.ds(q * CHUNK + k * LANES, LANES)]
                rowid_v[slot, pl.ds(k * LANES, LANES)] = (
                    ((iv >> 7) << 6) + ((iv >> 4) & 7) + foff)
                return carry2

            lax.fori_loop(0, VPC, rid, 0)
            pltpu.make_async_copy(
                tab_hbm.at[rowid_v.at[slot]],
                rows_v.at[slot],
                sem.at[slot],
            ).start()

        for d in range(DEPTH):
            fire(d)

        def chunk_body(j, carry):
            slot = j % DEPTH
            f = j // NCHUNK
            q = j % NCHUNK
            pltpu.make_async_copy(
                tab_hbm.at[pl.ds(0, CHUNK), :], rows_v.at[slot], sem.at[slot]
            ).wait()

            def sel(k, carry2):
                col = q * CHUNK + k * LANES
                iv = idx_v[f, pl.ds(col, LANES)]
                pos = lane_iota + k * LANES
                v = plsc.load_gather(rows_v.at[slot], [pos, iv & 15])
                acc = out_v[pl.ds(col, LANES)]
                out_v[pl.ds(col, LANES)] = acc + v
                return carry2

            lax.fori_loop(0, VPC, sel, 0)

            @pl.when(j + DEPTH < ngather)
            def _():
                fire(j + DEPTH)

            return carry

        lax.fori_loop(0, ngather, chunk_body, 0)

        pltpu.sync_copy(out_v, out_hbm.at[pl.ds(base, BPW)])

    return body


def _group_call(f0, nf, has_prev, idx_t, tab_g, aux):
    mesh = plsc.VectorSubcoreMesh(core_axis_name="c", subcore_axis_name="s")
    return pl.kernel(
        _make_body(f0, nf, has_prev),
        out_type=jax.ShapeDtypeStruct((B,), jnp.float32),
        mesh=mesh,
        scratch_types=[
            pltpu.VMEM((nf, BPW), jnp.int32),
            pltpu.VMEM((DEPTH, CHUNK), jnp.int32),
            pltpu.VMEM((DEPTH, CHUNK, LANES), jnp.float32),
            pltpu.VMEM((BPW,), jnp.float32),
            pltpu.VMEM((LANES,), jnp.float32),
            pltpu.SemaphoreType.DMA((DEPTH,)),
        ],
        compiler_params=pltpu.CompilerParams(
            needs_layout_passes=False, use_tc_tiling_on_sc=False),
        name=f"lookup_sum_f{f0}_{nf}",
    )(idx_t, tab_g, aux)


@jax.jit
def _lookup_sum(idx_t, tables, bias16):
    part = None
    f0 = 0
    tab_src = tables
    for nf in GROUPS:
        tab_pad = lax.pad(
            tab_src, jnp.float32(0),
            [(-f0, -(F - f0 - nf) + (GF - nf), 0), (0, VPAD - V, 0)])
        tab_src, _ = lax.optimization_barrier((tab_src, tab_pad))
        tab_g = (
            tab_pad.reshape(1, GF, CTILES, ROWLEN)
            .transpose(0, 2, 1, 3)
            .reshape(GROWS64, LANES)
        )
        aux = bias16 if part is None else part
        part = _group_call(f0, nf, part is not None, idx_t, tab_g, aux)
        f0 += nf
    return part


def kernel(indices, tables, bias):
    idx_t = indices.astype(jnp.int32).T
    bias16 = jnp.broadcast_to(bias.astype(jnp.float32), (LANES,))
    return _lookup_sum(idx_t, tables, bias16)


# speedup vs baseline: 1.4963x; 1.0253x over previous
import functools

import jax
import jax.numpy as jnp
from jax import lax
from jax.experimental import pallas as pl
from jax.experimental.pallas import tpu as pltpu
from jax.experimental.pallas import tpu_sc as plsc

B = 16384
F = 26
V = 1000000

NC = 2
NS = 16
NW = NC * NS
BPW = B // NW
LANES = 16
ROWLEN = 128
GF = 8
VPAD = 1000064
CTILES = VPAD // ROWLEN
GROWS64 = GF * CTILES * 8
CHUNK = 128
NCHUNK = BPW // CHUNK
VPC = CHUNK // LANES
DEPTH = 8

GROUPS = (8, 8, 8, 2)


def _make_body(f0, nf, has_prev):
    ngather = nf * NCHUNK

    def body(idx_hbm, tab_hbm, aux_hbm, out_hbm, idx_v, rowid_v, rows_v,
             out_v, bias_v, sem):
        wid = lax.axis_index("s") * NC + lax.axis_index("c")
        base = wid * BPW

        pltpu.sync_copy(idx_hbm.at[pl.ds(f0, nf), pl.ds(base, BPW)], idx_v)

        if has_prev:
            pltpu.sync_copy(aux_hbm.at[pl.ds(base, BPW)], out_v)
        else:
            pltpu.sync_copy(aux_hbm, bias_v)
            b_vec = bias_v[...]

            def init(c, carry):
                out_v[pl.ds(c * LANES, LANES)] = b_vec
                return carry

            lax.fori_loop(0, BPW // LANES, init, 0)

        lane_iota = lax.iota(jnp.int32, LANES)

        def fire(j):
            f = j // NCHUNK
            q = j % NCHUNK
            slot = j % DEPTH
            foff = f * 8

            def rid(k, carry2):
                iv = idx_v[f, pl.ds(q * CHUNK + k * LANES, LANES)]
                rowid_v[slot, pl.ds(k * LANES, LANES)] = (
                    ((iv >> 7) << 6) + ((iv >> 4) & 7) + foff)
                return carry2

            lax.fori_loop(0, VPC, rid, 0)
            pltpu.make_async_copy(
                tab_hbm.at[rowid_v.at[slot]],
                rows_v.at[slot],
                sem.at[slot],
            ).start()

        for d in range(DEPTH):
            fire(d)

        def chunk_body(j, carry):
            slot = j % DEPTH
            f = j // NCHUNK
            q = j % NCHUNK
            pltpu.make_async_copy(
                tab_hbm.at[pl.ds(0, CHUNK), :], rows_v.at[slot], sem.at[slot]
            ).wait()

            def sel(k, carry2):
                col = q * CHUNK + k * LANES
                iv = idx_v[f, pl.ds(col, LANES)]
                pos = lane_iota + k * LANES
                v = plsc.load_gather(rows_v.at[slot], [pos, iv & 15])
                acc = out_v[pl.ds(col, LANES)]
                out_v[pl.ds(col, LANES)] = acc + v
                return carry2

            lax.fori_loop(0, VPC, sel, 0)

            @pl.when(j + DEPTH < ngather)
            def _():
                fire(j + DEPTH)

            return carry

        lax.fori_loop(0, ngather, chunk_body, 0)

        pltpu.sync_copy(out_v, out_hbm.at[pl.ds(base, BPW)])

    return body


def _group_call(f0, nf, has_prev, idx_t, tab_g, aux):
    mesh = plsc.VectorSubcoreMesh(core_axis_name="c", subcore_axis_name="s")
    return pl.kernel(
        _make_body(f0, nf, has_prev),
        out_type=jax.ShapeDtypeStruct((B,), jnp.float32),
        mesh=mesh,
        scratch_types=[
            pltpu.VMEM((nf, BPW), jnp.int32),
            pltpu.VMEM((DEPTH, CHUNK), jnp.int32),
            pltpu.VMEM((DEPTH, CHUNK, LANES), jnp.float32),
            pltpu.VMEM((BPW,), jnp.float32),
            pltpu.VMEM((LANES,), jnp.float32),
            pltpu.SemaphoreType.DMA((DEPTH,)),
        ],
        compiler_params=pltpu.CompilerParams(
            needs_layout_passes=False, use_tc_tiling_on_sc=False),
        name=f"lookup_sum_f{f0}_{nf}",
    )(idx_t, tab_g, aux)


@jax.jit
def _lookup_sum(idx_t, tables, bias16):
    part = None
    f0 = 0
    tab_src = tables
    for nf in GROUPS:
        tab_pad = lax.pad(
            tab_src, jnp.float32(0),
            [(-f0, -(F - f0 - nf) + (GF - nf), 0), (0, VPAD - V, 0)])
        tab_src, _ = lax.optimization_barrier((tab_src, tab_pad))
        tab_g = (
            tab_pad.reshape(1, GF, CTILES, ROWLEN)
            .transpose(0, 2, 1, 3)
            .reshape(GROWS64, LANES)
        )
        aux = bias16 if part is None else part
        part = _group_call(f0, nf, part is not None, idx_t, tab_g, aux)
        f0 += nf
    return part


def kernel(indices, tables, bias):
    idx_t = indices.astype(jnp.int32).T
    bias16 = jnp.broadcast_to(bias.astype(jnp.float32), (LANES,))
    return _lookup_sum(idx_t, tables, bias16)
